# Initial kernel scaffold; baseline (speedup 1.0000x reference)
import jax
import jax.numpy as jnp
from jax.experimental import pallas as pl


def kernel(node_embs, edge_index, edge_type, rel_weight):
    raise NotImplementedError("write your pallas kernel here")


# speedup vs baseline: 1.0865x; 1.0865x over previous
import functools

import jax
import jax.numpy as jnp
from jax import lax
from jax.experimental import pallas as pl
from jax.experimental.pallas import tpu as pltpu
from jax.experimental.pallas import tpu_sc as plsc

_E_SIZE = 6884
_NE = 320000
_D = 128
_L = 16

_NC = 2
_NS = 16
_NW = _NC * _NS
_EPW = _NE // _NW
_B = 80
_NCHUNK = _EPW // _B


def _sqrt_nr(x):
    i = lax.bitcast_convert_type(x, jnp.int32)
    y = lax.bitcast_convert_type(jnp.int32(0x5F3759DF) - (i >> 1), jnp.float32)
    for _ in range(4):
        y = y * (1.5 - 0.5 * x * y * y)
    return x * y


def _tile_body(node_hbm, rel_hbm, src_hbm, tgt_hbm, nsrc_hbm, ntgt_hbm, et_hbm,
               out_hbm,
               src_i, tgt_i, nsrc_i, ntgt_i, et_i,
               s_v, t_v, ns_v, nt_v, r_v, acc_v, sem):
    wid = lax.axis_index("s") * _NC + lax.axis_index("c")
    base = pl.multiple_of(wid * _EPW, 8)

    pltpu.sync_copy(src_hbm.at[pl.ds(base, _EPW)], src_i)
    pltpu.sync_copy(tgt_hbm.at[pl.ds(base, _EPW)], tgt_i)
    pltpu.sync_copy(nsrc_hbm.at[pl.ds(base, _EPW)], nsrc_i)
    pltpu.sync_copy(ntgt_hbm.at[pl.ds(base, _EPW)], ntgt_i)
    pltpu.sync_copy(et_hbm.at[pl.ds(base, _EPW)], et_i)

    def chunk_body(c, loss):
        off = pl.multiple_of(c * _B, 8)
        c1 = pltpu.async_copy(node_hbm.at[src_i.at[pl.ds(off, _B)]], s_v, sem)
        c2 = pltpu.async_copy(node_hbm.at[tgt_i.at[pl.ds(off, _B)]], t_v, sem)
        c3 = pltpu.async_copy(node_hbm.at[nsrc_i.at[pl.ds(off, _B)]], ns_v, sem)
        c4 = pltpu.async_copy(node_hbm.at[ntgt_i.at[pl.ds(off, _B)]], nt_v, sem)
        c5 = pltpu.async_copy(rel_hbm.at[et_i.at[pl.ds(off, _B)]], r_v, sem)
        c1.wait(); c2.wait(); c3.wait(); c4.wait(); c5.wait()

        def grp_body(g, loss2):
            idx_e = g * _L + lax.iota(jnp.int32, _L)

            def dim_body(j, carry):
                accp, accn = carry
                idx_j = jnp.full((_L,), 0, jnp.int32) + j
                rv = plsc.load_gather(r_v, [idx_e, idx_j])
                d = plsc.load_gather(s_v, [idx_e, idx_j]) + rv \
                    - plsc.load_gather(t_v, [idx_e, idx_j])
                dn = plsc.load_gather(ns_v, [idx_e, idx_j]) + rv \
                    - plsc.load_gather(nt_v, [idx_e, idx_j])
                return accp + d * d, accn + dn * dn

            zero = jnp.zeros((_L,), jnp.float32)
            accp, accn = lax.fori_loop(0, _D, dim_body, (zero, zero), unroll=8)
            pos = _sqrt_nr(accp)
            neg = _sqrt_nr(accn)
            return loss2 + jnp.maximum(pos - neg + 1.0, 0.0)

        return lax.fori_loop(0, _B // _L, grp_body, loss, unroll=False)

    zero16 = jnp.zeros((_L,), jnp.float32)
    total = lax.fori_loop(0, _NCHUNK, chunk_body, zero16, unroll=False)

    acc_v[...] = total
    pltpu.sync_copy(acc_v, out_hbm.at[wid])


@functools.partial(jax.jit, static_argnames=())
def _loss_sc(node_embs, rel_weight, src, tgt, nsrc, ntgt, et):
    mesh = plsc.VectorSubcoreMesh(core_axis_name="c", subcore_axis_name="s")
    run = functools.partial(
        pl.kernel,
        mesh=mesh,
        compiler_params=pltpu.CompilerParams(needs_layout_passes=False),
        out_type=jax.ShapeDtypeStruct((_NW, _L), jnp.float32),
        scratch_types=[
            pltpu.VMEM((_EPW,), jnp.int32),
            pltpu.VMEM((_EPW,), jnp.int32),
            pltpu.VMEM((_EPW,), jnp.int32),
            pltpu.VMEM((_EPW,), jnp.int32),
            pltpu.VMEM((_EPW,), jnp.int32),
            pltpu.VMEM((_B, _D), jnp.float32),
            pltpu.VMEM((_B, _D), jnp.float32),
            pltpu.VMEM((_B, _D), jnp.float32),
            pltpu.VMEM((_B, _D), jnp.float32),
            pltpu.VMEM((_B, _D), jnp.float32),
            pltpu.VMEM((_L,), jnp.float32),
            pltpu.SemaphoreType.DMA,
        ],
    )(_tile_body)
    return run(node_embs, rel_weight, src, tgt, nsrc, ntgt, et)


def kernel(node_embs, edge_index, edge_type, rel_weight):
    neg_edge_index = jax.random.randint(
        jax.random.key(42), edge_index.shape, 0, _E_SIZE, dtype=edge_index.dtype)
    partials = _loss_sc(
        node_embs, rel_weight,
        edge_index[0], edge_index[1],
        neg_edge_index[0], neg_edge_index[1],
        edge_type,
    )
    return jnp.sum(partials) / jnp.float32(_NE)


# speedup vs baseline: 5.4860x; 5.0491x over previous
import functools

import jax
import jax.numpy as jnp
from jax import lax
from jax.experimental import pallas as pl
from jax.experimental.pallas import tpu as pltpu
from jax.experimental.pallas import tpu_sc as plsc

_E_SIZE = 6884
_NE = 320000
_D = 128
_L = 16

_NC = 2
_NS = 16
_NW = _NC * _NS
_EPW = _NE // _NW
_B = 80
_NCHUNK = _EPW // _B


def _sqrt_nr(x):
    i = lax.bitcast_convert_type(x, jnp.int32)
    y = lax.bitcast_convert_type(jnp.int32(0x5F3759DF) - (i >> 1), jnp.float32)
    for _ in range(4):
        y = y * (1.5 - 0.5 * x * y * y)
    return x * y


def _tile_body(node_hbm, rel_hbm, src_hbm, tgt_hbm, nsrc_hbm, ntgt_hbm, et_hbm,
               out_hbm,
               src_i, tgt_i, nsrc_i, ntgt_i, et_i,
               s_v, t_v, ns_v, nt_v, r_v, acc_v, sem):
    wid = lax.axis_index("s") * _NC + lax.axis_index("c")
    base = pl.multiple_of(wid * _EPW, 8)

    pltpu.sync_copy(src_hbm.at[pl.ds(base, _EPW)], src_i)
    pltpu.sync_copy(tgt_hbm.at[pl.ds(base, _EPW)], tgt_i)
    pltpu.sync_copy(nsrc_hbm.at[pl.ds(base, _EPW)], nsrc_i)
    pltpu.sync_copy(ntgt_hbm.at[pl.ds(base, _EPW)], ntgt_i)
    pltpu.sync_copy(et_hbm.at[pl.ds(base, _EPW)], et_i)

    def chunk_body(c, loss):
        off = pl.multiple_of(c * _B, 8)
        c1 = pltpu.async_copy(node_hbm.at[src_i.at[pl.ds(off, _B)]], s_v, sem)
        c2 = pltpu.async_copy(node_hbm.at[tgt_i.at[pl.ds(off, _B)]], t_v, sem)
        c3 = pltpu.async_copy(node_hbm.at[nsrc_i.at[pl.ds(off, _B)]], ns_v, sem)
        c4 = pltpu.async_copy(node_hbm.at[ntgt_i.at[pl.ds(off, _B)]], nt_v, sem)
        c5 = pltpu.async_copy(rel_hbm.at[et_i.at[pl.ds(off, _B)]], r_v, sem)
        c1.wait(); c2.wait(); c3.wait(); c4.wait(); c5.wait()

        def edge_body(e, loss2):
            accp = None
            accn = None
            for j in range(_D // _L):
                sl = pl.ds(j * _L, _L)
                rv = r_v[e, sl]
                d = s_v[e, sl] + rv - t_v[e, sl]
                dn = ns_v[e, sl] + rv - nt_v[e, sl]
                accp = d * d if accp is None else accp + d * d
                accn = dn * dn if accn is None else accn + dn * dn
            pos = _sqrt_nr(jnp.sum(accp))
            neg = _sqrt_nr(jnp.sum(accn))
            return loss2 + jnp.maximum(pos - neg + 1.0, 0.0)

        return lax.fori_loop(0, _B, edge_body, loss, unroll=2)

    total = lax.fori_loop(0, _NCHUNK, chunk_body, jnp.float32(0.0),
                          unroll=False)

    acc_v[...] = jnp.where(lax.iota(jnp.int32, _L) == 0, total, 0.0)
    pltpu.sync_copy(acc_v, out_hbm.at[wid])


@functools.partial(jax.jit, static_argnames=())
def _loss_sc(node_embs, rel_weight, src, tgt, nsrc, ntgt, et):
    mesh = plsc.VectorSubcoreMesh(core_axis_name="c", subcore_axis_name="s")
    run = functools.partial(
        pl.kernel,
        mesh=mesh,
        compiler_params=pltpu.CompilerParams(needs_layout_passes=False),
        out_type=jax.ShapeDtypeStruct((_NW, _L), jnp.float32),
        scratch_types=[
            pltpu.VMEM((_EPW,), jnp.int32),
            pltpu.VMEM((_EPW,), jnp.int32),
            pltpu.VMEM((_EPW,), jnp.int32),
            pltpu.VMEM((_EPW,), jnp.int32),
            pltpu.VMEM((_EPW,), jnp.int32),
            pltpu.VMEM((_B, _D), jnp.float32),
            pltpu.VMEM((_B, _D), jnp.float32),
            pltpu.VMEM((_B, _D), jnp.float32),
            pltpu.VMEM((_B, _D), jnp.float32),
            pltpu.VMEM((_B, _D), jnp.float32),
            pltpu.VMEM((_L,), jnp.float32),
            pltpu.SemaphoreType.DMA,
        ],
    )(_tile_body)
    return run(node_embs, rel_weight, src, tgt, nsrc, ntgt, et)


def kernel(node_embs, edge_index, edge_type, rel_weight):
    neg_edge_index = jax.random.randint(
        jax.random.key(42), edge_index.shape, 0, _E_SIZE, dtype=edge_index.dtype)
    partials = _loss_sc(
        node_embs, rel_weight,
        edge_index[0], edge_index[1],
        neg_edge_index[0], neg_edge_index[1],
        edge_type,
    )
    return jnp.sum(partials) / jnp.float32(_NE)


# speedup vs baseline: 6.8469x; 1.2481x over previous
import functools

import jax
import jax.numpy as jnp
from jax import lax
from jax.experimental import pallas as pl
from jax.experimental.pallas import tpu as pltpu
from jax.experimental.pallas import tpu_sc as plsc

_E_SIZE = 6884
_NE = 320000
_D = 128
_L = 16

_NC = 2
_NS = 16
_NW = _NC * _NS
_EPW = _NE // _NW
_B = 40
_NCHUNK = _EPW // _B
_NT = 5


def _sqrt_nr(x):
    i = lax.bitcast_convert_type(x, jnp.int32)
    y = lax.bitcast_convert_type(jnp.int32(0x5F3759DF) - (i >> 1), jnp.float32)
    for _ in range(3):
        y = y * (1.5 - 0.5 * x * y * y)
    return x * y


def _tile_body(node_hbm, rel_hbm, src_hbm, tgt_hbm, nsrc_hbm, ntgt_hbm, et_hbm,
               out_hbm,
               src_i, tgt_i, nsrc_i, ntgt_i, et_i,
               rows_v, acc_v, sem0, sem1):
    wid = lax.axis_index("s") * _NC + lax.axis_index("c")
    base = pl.multiple_of(wid * _EPW, 8)

    pltpu.sync_copy(src_hbm.at[pl.ds(base, _EPW)], src_i)
    pltpu.sync_copy(tgt_hbm.at[pl.ds(base, _EPW)], tgt_i)
    pltpu.sync_copy(nsrc_hbm.at[pl.ds(base, _EPW)], nsrc_i)
    pltpu.sync_copy(ntgt_hbm.at[pl.ds(base, _EPW)], ntgt_i)
    pltpu.sync_copy(et_hbm.at[pl.ds(base, _EPW)], et_i)

    sems = (sem0, sem1)

    def fire(c, p):
        off = pl.multiple_of(c * _B, 8)
        sem = sems[p]
        pltpu.async_copy(node_hbm.at[src_i.at[pl.ds(off, _B)]],
                         rows_v.at[p, pl.ds(0 * _B, _B)], sem)
        pltpu.async_copy(node_hbm.at[tgt_i.at[pl.ds(off, _B)]],
                         rows_v.at[p, pl.ds(1 * _B, _B)], sem)
        pltpu.async_copy(node_hbm.at[nsrc_i.at[pl.ds(off, _B)]],
                         rows_v.at[p, pl.ds(2 * _B, _B)], sem)
        pltpu.async_copy(node_hbm.at[ntgt_i.at[pl.ds(off, _B)]],
                         rows_v.at[p, pl.ds(3 * _B, _B)], sem)
        pltpu.async_copy(rel_hbm.at[et_i.at[pl.ds(off, _B)]],
                         rows_v.at[p, pl.ds(4 * _B, _B)], sem)

    def drain(p):
        pltpu.make_async_copy(node_hbm.at[pl.ds(0, _NT * _B)],
                              rows_v.at[p], sems[p]).wait()

    def compute(p, loss):
        def edge_body(e, loss2):
            accp = None
            accn = None
            for j in range(_D // _L):
                sl = pl.ds(j * _L, _L)
                rv = rows_v[p, 4 * _B + e, sl]
                d = rows_v[p, e, sl] + rv - rows_v[p, _B + e, sl]
                dn = rows_v[p, 2 * _B + e, sl] + rv - rows_v[p, 3 * _B + e, sl]
                accp = d * d if accp is None else accp + d * d
                accn = dn * dn if accn is None else accn + dn * dn
            pos = _sqrt_nr(jnp.sum(accp))
            neg = _sqrt_nr(jnp.sum(accn))
            return loss2 + jnp.maximum(pos - neg + 1.0, 0.0)

        return lax.fori_loop(0, _B, edge_body, loss, unroll=4)

    fire(0, 0)

    def pair_body(i, loss):
        fire(2 * i + 1, 1)
        drain(0)
        loss = compute(0, loss)

        @pl.when(i < _NCHUNK // 2 - 1)
        def _():
            fire(2 * i + 2, 0)

        drain(1)
        return compute(1, loss)

    total = lax.fori_loop(0, _NCHUNK // 2, pair_body, jnp.float32(0.0),
                          unroll=False)

    acc_v[...] = jnp.where(lax.iota(jnp.int32, _L) == 0, total, 0.0)
    pltpu.sync_copy(acc_v, out_hbm.at[wid])


@functools.partial(jax.jit, static_argnames=())
def _loss_sc(node_embs, rel_weight, src, tgt, nsrc, ntgt, et):
    mesh = plsc.VectorSubcoreMesh(core_axis_name="c", subcore_axis_name="s")
    run = functools.partial(
        pl.kernel,
        mesh=mesh,
        compiler_params=pltpu.CompilerParams(needs_layout_passes=False),
        out_type=jax.ShapeDtypeStruct((_NW, _L), jnp.float32),
        scratch_types=[
            pltpu.VMEM((_EPW,), jnp.int32),
            pltpu.VMEM((_EPW,), jnp.int32),
            pltpu.VMEM((_EPW,), jnp.int32),
            pltpu.VMEM((_EPW,), jnp.int32),
            pltpu.VMEM((_EPW,), jnp.int32),
            pltpu.VMEM((2, _NT * _B, _D), jnp.float32),
            pltpu.VMEM((_L,), jnp.float32),
            pltpu.SemaphoreType.DMA,
            pltpu.SemaphoreType.DMA,
        ],
    )(_tile_body)
    return run(node_embs, rel_weight, src, tgt, nsrc, ntgt, et)


def kernel(node_embs, edge_index, edge_type, rel_weight):
    neg_edge_index = jax.random.randint(
        jax.random.key(42), edge_index.shape, 0, _E_SIZE, dtype=edge_index.dtype)
    partials = _loss_sc(
        node_embs, rel_weight,
        edge_index[0], edge_index[1],
        neg_edge_index[0], neg_edge_index[1],
        edge_type,
    )
    return jnp.sum(partials) / jnp.float32(_NE)


# speedup vs baseline: 7.5160x; 1.0977x over previous
import functools

import jax
import jax.numpy as jnp
from jax import lax
from jax.experimental import pallas as pl
from jax.experimental.pallas import tpu as pltpu
from jax.experimental.pallas import tpu_sc as plsc

_E_SIZE = 6884
_NE = 320000
_D = 128
_L = 16

_NC = 2
_NS = 16
_NW = _NC * _NS
_EPW = _NE // _NW
_B = 40
_NCHUNK = _EPW // _B
_NT = 5


def _sqrt_nr(x):
    i = lax.bitcast_convert_type(x, jnp.int32)
    y = lax.bitcast_convert_type(jnp.int32(0x5F3759DF) - (i >> 1), jnp.float32)
    for _ in range(3):
        y = y * (1.5 - 0.5 * x * y * y)
    return x * y


def _tile_body(node_hbm, rel_hbm, src_hbm, tgt_hbm, nsrc_hbm, ntgt_hbm, et_hbm,
               out_hbm,
               src_i, tgt_i, nsrc_i, ntgt_i, et_i,
               rows_v, acc_v, sem0, sem1):
    wid = lax.axis_index("s") * _NC + lax.axis_index("c")
    base = pl.multiple_of(wid * _EPW, 8)

    pltpu.sync_copy(src_hbm.at[pl.ds(base, _EPW)], src_i)
    pltpu.sync_copy(tgt_hbm.at[pl.ds(base, _EPW)], tgt_i)
    pltpu.sync_copy(nsrc_hbm.at[pl.ds(base, _EPW)], nsrc_i)
    pltpu.sync_copy(ntgt_hbm.at[pl.ds(base, _EPW)], ntgt_i)
    pltpu.sync_copy(et_hbm.at[pl.ds(base, _EPW)], et_i)

    sems = (sem0, sem1)

    def fire(c, p):
        off = pl.multiple_of(c * _B, 8)
        sem = sems[p]
        pltpu.async_copy(node_hbm.at[src_i.at[pl.ds(off, _B)]],
                         rows_v.at[p, pl.ds(0 * _B, _B)], sem)
        pltpu.async_copy(node_hbm.at[tgt_i.at[pl.ds(off, _B)]],
                         rows_v.at[p, pl.ds(1 * _B, _B)], sem)
        pltpu.async_copy(node_hbm.at[nsrc_i.at[pl.ds(off, _B)]],
                         rows_v.at[p, pl.ds(2 * _B, _B)], sem)
        pltpu.async_copy(node_hbm.at[ntgt_i.at[pl.ds(off, _B)]],
                         rows_v.at[p, pl.ds(3 * _B, _B)], sem)
        pltpu.async_copy(rel_hbm.at[et_i.at[pl.ds(off, _B)]],
                         rows_v.at[p, pl.ds(4 * _B, _B)], sem)

    def drain(p):
        pltpu.make_async_copy(node_hbm.at[pl.ds(0, _NT * _B)],
                              rows_v.at[p], sems[p]).wait()

    def compute(p, loss):
        def up(v):
            return plsc.unpack(v, format=plsc.PackFormat.INTERLEAVED)

        def edge_body(e, loss2):
            accp = None
            accn = None
            for j in range(_D // (2 * _L)):
                sl = pl.ds(j * 2 * _L, 2 * _L)
                rv0, rv1 = up(rows_v[p, 4 * _B + e, sl])
                s0, s1 = up(rows_v[p, e, sl])
                t0, t1 = up(rows_v[p, _B + e, sl])
                ns0, ns1 = up(rows_v[p, 2 * _B + e, sl])
                nt0, nt1 = up(rows_v[p, 3 * _B + e, sl])
                d0 = s0 + rv0 - t0
                d1 = s1 + rv1 - t1
                dn0 = ns0 + rv0 - nt0
                dn1 = ns1 + rv1 - nt1
                if accp is None:
                    accp = d0 * d0 + d1 * d1
                    accn = dn0 * dn0 + dn1 * dn1
                else:
                    accp = accp + d0 * d0 + d1 * d1
                    accn = accn + dn0 * dn0 + dn1 * dn1
            pos = _sqrt_nr(jnp.sum(accp))
            neg = _sqrt_nr(jnp.sum(accn))
            return loss2 + jnp.maximum(pos - neg + 1.0, 0.0)

        return lax.fori_loop(0, _B, edge_body, loss, unroll=4)

    fire(0, 0)

    def pair_body(i, loss):
        fire(2 * i + 1, 1)
        drain(0)
        loss = compute(0, loss)

        @pl.when(i < _NCHUNK // 2 - 1)
        def _():
            fire(2 * i + 2, 0)

        drain(1)
        return compute(1, loss)

    total = lax.fori_loop(0, _NCHUNK // 2, pair_body, jnp.float32(0.0),
                          unroll=False)

    acc_v[...] = jnp.where(lax.iota(jnp.int32, _L) == 0, total, 0.0)
    pltpu.sync_copy(acc_v, out_hbm.at[wid])


@functools.partial(jax.jit, static_argnames=())
def _loss_sc(node_embs, rel_weight, src, tgt, nsrc, ntgt, et):
    mesh = plsc.VectorSubcoreMesh(core_axis_name="c", subcore_axis_name="s")
    run = functools.partial(
        pl.kernel,
        mesh=mesh,
        compiler_params=pltpu.CompilerParams(needs_layout_passes=False,
                                             use_tc_tiling_on_sc=False),
        out_type=jax.ShapeDtypeStruct((_NW, _L), jnp.float32),
        scratch_types=[
            pltpu.VMEM((_EPW,), jnp.int32),
            pltpu.VMEM((_EPW,), jnp.int32),
            pltpu.VMEM((_EPW,), jnp.int32),
            pltpu.VMEM((_EPW,), jnp.int32),
            pltpu.VMEM((_EPW,), jnp.int32),
            pltpu.VMEM((2, _NT * _B, _D), jnp.bfloat16),
            pltpu.VMEM((_L,), jnp.float32),
            pltpu.SemaphoreType.DMA,
            pltpu.SemaphoreType.DMA,
        ],
    )(_tile_body)
    return run(node_embs, rel_weight, src, tgt, nsrc, ntgt, et)


def kernel(node_embs, edge_index, edge_type, rel_weight):
    neg_edge_index = jax.random.randint(
        jax.random.key(42), edge_index.shape, 0, _E_SIZE, dtype=edge_index.dtype)
    partials = _loss_sc(
        node_embs.astype(jnp.bfloat16), rel_weight.astype(jnp.bfloat16),
        edge_index[0], edge_index[1],
        neg_edge_index[0], neg_edge_index[1],
        edge_type,
    )
    return jnp.sum(partials) / jnp.float32(_NE)
